# baseline (speedup 1.0000x reference)
import dataclasses
import functools

import jax
import jax.numpy as jnp
from jax import lax
from jax.experimental import pallas as pl
from jax.experimental.pallas import tpu as pltpu
from jax.experimental.pallas import tpu_sc as plsc

N = 10000
E = 160000
D = 256

NPAD = 10240
HALFN = 5120
ACC_R = 5136
CHUNK = 128
NCH = E // CHUNK
NSUB = 16
HIST = 10496
STRIPE = HIST // NSUB
BR = 1024
GRID = NPAD // BR

_mesh = plsc.VectorSubcoreMesh(core_axis_name="c", subcore_axis_name="s")

_cp = pltpu.CompilerParams()
if "needs_layout_passes" in pltpu.CompilerParams.__dataclass_fields__:
    _cp = dataclasses.replace(_cp, needs_layout_passes=False)
if "use_tc_tiling_on_sc" in pltpu.CompilerParams.__dataclass_fields__:
    _cp = dataclasses.replace(_cp, use_tc_tiling_on_sc=False)


@functools.partial(
    pl.kernel,
    mesh=_mesh,
    out_type=jax.ShapeDtypeStruct((HIST,), jnp.float32),
    scratch_types=[
        pltpu.VMEM((CHUNK,), jnp.int32),
        pltpu.VMEM((HIST,), jnp.float32),
        pltpu.VMEM((NSUB * STRIPE,), jnp.float32),
        pltpu.VMEM_SHARED((NSUB * HIST,), jnp.float32),
    ],
    compiler_params=_cp,
)
def _deg_sc(dst_hbm, deg_hbm, dst_v, hist_v, red_v, part_sh):
    c = lax.axis_index("c")
    s = lax.axis_index("s")
    one16 = jnp.full((16,), 1.0, jnp.float32)
    zero16 = jnp.zeros((16,), jnp.float32)

    @pl.loop(0, HIST // 16)
    def _(j):
        hist_v[pl.ds(j * 16, 16)] = zero16

    @pl.loop(0, (NCH + NSUB - 1) // NSUB)
    def _(i):
        k = i * NSUB + s

        @pl.when(k < NCH)
        def _():
            pltpu.sync_copy(dst_hbm.at[pl.ds(k * CHUNK, CHUNK)], dst_v)
            for j in range(CHUNK // 16):
                d16 = dst_v[pl.ds(j * 16, 16)]
                plsc.addupdate_scatter(hist_v, [d16], one16)

    pltpu.sync_copy(hist_v, part_sh.at[pl.ds(s * HIST, HIST)])
    plsc.subcore_barrier()
    for k in range(NSUB):
        pltpu.sync_copy(part_sh.at[pl.ds(k * HIST + s * STRIPE, STRIPE)],
                        red_v.at[pl.ds(k * STRIPE, STRIPE)])
    for j in range(STRIPE // 16):
        tot = red_v[pl.ds(j * 16, 16)]
        for k in range(1, NSUB):
            tot = tot + red_v[pl.ds(k * STRIPE + j * 16, 16)]
        hist_v[pl.ds(j * 16, 16)] = tot

    @pl.when(((s < 8) & (c == 0)) | ((s >= 8) & (c == 1)))
    def _():
        pltpu.sync_copy(hist_v.at[pl.ds(0, STRIPE)],
                        deg_hbm.at[pl.ds(s * STRIPE, STRIPE)])


SUP = 1280
NSUP = E // SUP
CPS = SUP // CHUNK


@functools.partial(
    pl.kernel,
    mesh=_mesh,
    out_type=jax.ShapeDtypeStruct((NPAD, 16, 16), jnp.float32),
    scratch_types=[
        pltpu.VMEM((2, SUP), jnp.int32),
        pltpu.VMEM((2, CHUNK), jnp.int32),
        pltpu.VMEM((2, CHUNK), jnp.int32),
        pltpu.VMEM((2, CHUNK, 16), jnp.float32),
        pltpu.VMEM((ACC_R, 16), jnp.float32),
        pltpu.SemaphoreType.DMA,
        pltpu.SemaphoreType.DMA,
    ],
    compiler_params=_cp,
)
def _mp_sc(y16_hbm, eb_hbm, out_hbm,
           ebuf, gidx_v, ridx_v, msg_v, acc_v, sem0, sem1):
    c = lax.axis_index("c")
    s = lax.axis_index("s")
    lo = c * HALFN
    zero16 = jnp.zeros((16,), jnp.float32)
    iota16 = lax.iota(jnp.int32, 16)
    sems = (sem0, sem1)

    @pl.loop(0, ACC_R)
    def _(r):
        acc_v[r, :] = zero16

    def prep(t, b):
        for g in range(CHUNK // 16):
            s16 = ebuf[0, pl.ds(t * CHUNK + g * 16, 16)]
            d16 = ebuf[1, pl.ds(t * CHUNK + g * 16, 16)]
            local = d16 - lo
            ok = (local >= 0) & (local < HALFN)
            ridx_v[b, pl.ds(g * 16, 16)] = jnp.where(ok, local, HALFN)
            gidx_v[b, pl.ds(g * 16, 16)] = jnp.where(ok, s16 * 16 + s, -1)

    def fire(b):
        idx = plsc.Indices(gidx_v.at[b], ignored_value=-1)
        return pltpu.async_copy(y16_hbm.at[idx], msg_v.at[b], sems[b])

    def process(b):
        rows = [ridx_v[b, pl.ds(g * 16, 16)] for g in range(CHUNK // 16)]
        mrows = [iota16 + (g * 16) for g in range(CHUNK // 16)]

        @plsc.parallel_loop(0, 16, unroll=4)
        def _(j):
            colj = jnp.full((16,), j, jnp.int32)
            for g in range(CHUNK // 16):
                vals = plsc.load_gather(msg_v.at[b], [mrows[g], colj])
                plsc.addupdate_scatter(acc_v, [rows[g], colj], vals)

    @pl.loop(0, NSUP)
    def _(u):
        pltpu.sync_copy(eb_hbm.at[u], ebuf)
        prep(0, 0)
        d = fire(0)
        for t in range(1, CPS):
            b = t % 2
            prep(t, b)
            d_next = fire(b)
            d.wait()
            process(1 - b)
            d = d_next
        d.wait()
        process((CPS - 1) % 2)

    pltpu.sync_copy(acc_v.at[pl.ds(0, HALFN)], out_hbm.at[pl.ds(lo, HALFN), s])


def _row_spec():
    return pl.BlockSpec((BR, D), lambda i: (i, 0))


def _col_spec():
    return pl.BlockSpec((BR, 1), lambda i: (i, 0))


def _w_spec():
    return pl.BlockSpec((D, D), lambda i: (0, 0))


def _vec_spec():
    return pl.BlockSpec((1, D), lambda i: (0, 0))


def _layer_norm(h, g, b):
    mu = jnp.mean(h, axis=-1, keepdims=True)
    var = jnp.mean((h - mu) ** 2, axis=-1, keepdims=True)
    return (h - mu) * lax.rsqrt(var + 1e-5) * g + b


def _tc1_body(x_ref, w_ref, deg_ref, y_ref, dinv_ref):
    deg = deg_ref[...] + 1.0
    dinv = jnp.where(deg > 0, lax.rsqrt(deg), 0.0)
    dinv_ref[...] = dinv
    y_ref[...] = dinv * jnp.dot(x_ref[...], w_ref[...],
                                preferred_element_type=jnp.float32)


_tc1 = pl.pallas_call(
    _tc1_body,
    grid=(GRID,),
    in_specs=[_row_spec(), _w_spec(), _col_spec()],
    out_specs=[_row_spec(), _col_spec()],
    out_shape=[jax.ShapeDtypeStruct((NPAD, D), jnp.float32),
               jax.ShapeDtypeStruct((NPAD, 1), jnp.float32)],
)


def _tc2_body(acc_ref, y_ref, dinv_ref, b_ref, g_ref, bln_ref, w_ref,
              h_ref, y1_ref):
    dinv = dinv_ref[...]
    h = jnp.maximum(dinv * (acc_ref[...] + y_ref[...]) + b_ref[...], 0.0)
    h_ref[...] = h
    t = _layer_norm(h, g_ref[...], bln_ref[...])
    y1_ref[...] = dinv * jnp.dot(t, w_ref[...],
                                 preferred_element_type=jnp.float32)


_tc2 = pl.pallas_call(
    _tc2_body,
    grid=(GRID,),
    in_specs=[_row_spec(), _row_spec(), _col_spec(), _vec_spec(), _vec_spec(),
              _vec_spec(), _w_spec()],
    out_specs=[_row_spec(), _row_spec()],
    out_shape=[jax.ShapeDtypeStruct((NPAD, D), jnp.float32),
               jax.ShapeDtypeStruct((NPAD, D), jnp.float32)],
)


def _tc3_body(acc_ref, y_ref, dinv_ref, b_ref, idn_ref, w_ref, y2_ref):
    dinv = dinv_ref[...]
    h = jnp.maximum(dinv * (acc_ref[...] + y_ref[...]) + b_ref[...], 0.0)
    h = h + idn_ref[...]
    y2_ref[...] = dinv * jnp.dot(h, w_ref[...],
                                 preferred_element_type=jnp.float32)


_tc3 = pl.pallas_call(
    _tc3_body,
    grid=(GRID,),
    in_specs=[_row_spec(), _row_spec(), _col_spec(), _vec_spec(), _row_spec(),
              _w_spec()],
    out_specs=_row_spec(),
    out_shape=jax.ShapeDtypeStruct((NPAD, D), jnp.float32),
)


def _tc4_body(acc_ref, y_ref, dinv_ref, b_ref, g_ref, bln_ref, o_ref):
    o = dinv_ref[...] * (acc_ref[...] + y_ref[...]) + b_ref[...]
    o_ref[...] = _layer_norm(o, g_ref[...], bln_ref[...])


_tc4 = pl.pallas_call(
    _tc4_body,
    grid=(GRID,),
    in_specs=[_row_spec(), _row_spec(), _col_spec(), _vec_spec(), _vec_spec(),
              _vec_spec()],
    out_specs=_row_spec(),
    out_shape=jax.ShapeDtypeStruct((NPAD, D), jnp.float32),
)


def _mp(y, eb):
    acc3 = _mp_sc(y.reshape(NPAD * 16, 16), eb)
    return acc3.reshape(NPAD, D)


def kernel(x, edge_index, W0, b0, W1, b1, W2, b2, ln0_g, ln0_b, fn_g, fn_b):
    src = edge_index[0]
    dst = edge_index[1]
    eb = jnp.stack([src.reshape(NSUP, SUP), dst.reshape(NSUP, SUP)], axis=1)
    xp = jnp.pad(x, ((0, NPAD - N), (0, 0)))

    deg = _deg_sc(dst)[:NPAD].reshape(NPAD, 1)
    y0, dinv = _tc1(xp, W0, deg)
    acc0 = _mp(y0, eb)
    h, y1 = _tc2(acc0, y0, dinv, b0.reshape(1, D), ln0_g.reshape(1, D),
                 ln0_b.reshape(1, D), W1)
    acc1 = _mp(y1, eb)
    y2 = _tc3(acc1, y1, dinv, b1.reshape(1, D), h, W2)
    acc2 = _mp(y2, eb)
    out = _tc4(acc2, y2, dinv, b2.reshape(1, D), fn_g.reshape(1, D),
               fn_b.reshape(1, D))
    return out[:N]


# speedup vs baseline: 1.9902x; 1.9902x over previous
import dataclasses
import functools

import jax
import jax.numpy as jnp
from jax import lax
from jax.experimental import pallas as pl
from jax.experimental.pallas import tpu as pltpu
from jax.experimental.pallas import tpu_sc as plsc

N = 10000
E = 160000
D = 256

NPAD = 10240
HALFN = 5120
ACC_R = 5136
CHUNK = 128
NCH = E // CHUNK
NSUB = 16
HIST = 10496
STRIPE = HIST // NSUB
BR = 1024
GRID = NPAD // BR

_mesh = plsc.VectorSubcoreMesh(core_axis_name="c", subcore_axis_name="s")

_cp = pltpu.CompilerParams()
if "needs_layout_passes" in pltpu.CompilerParams.__dataclass_fields__:
    _cp = dataclasses.replace(_cp, needs_layout_passes=False)
if "use_tc_tiling_on_sc" in pltpu.CompilerParams.__dataclass_fields__:
    _cp = dataclasses.replace(_cp, use_tc_tiling_on_sc=False)


@functools.partial(
    pl.kernel,
    mesh=_mesh,
    out_type=jax.ShapeDtypeStruct((HIST,), jnp.float32),
    scratch_types=[
        pltpu.VMEM((CHUNK,), jnp.int32),
        pltpu.VMEM((HIST,), jnp.float32),
        pltpu.VMEM((NSUB * STRIPE,), jnp.float32),
        pltpu.VMEM_SHARED((NSUB * HIST,), jnp.float32),
    ],
    compiler_params=_cp,
)
def _deg_sc(dst_hbm, deg_hbm, dst_v, hist_v, red_v, part_sh):
    c = lax.axis_index("c")
    s = lax.axis_index("s")
    one16 = jnp.full((16,), 1.0, jnp.float32)
    zero16 = jnp.zeros((16,), jnp.float32)

    @pl.loop(0, HIST // 16)
    def _(j):
        hist_v[pl.ds(j * 16, 16)] = zero16

    @pl.loop(0, (NCH + NSUB - 1) // NSUB)
    def _(i):
        k = i * NSUB + s

        @pl.when(k < NCH)
        def _():
            pltpu.sync_copy(dst_hbm.at[pl.ds(k * CHUNK, CHUNK)], dst_v)
            for j in range(CHUNK // 16):
                d16 = dst_v[pl.ds(j * 16, 16)]
                plsc.addupdate_scatter(hist_v, [d16], one16)

    pltpu.sync_copy(hist_v, part_sh.at[pl.ds(s * HIST, HIST)])
    plsc.subcore_barrier()
    for k in range(NSUB):
        pltpu.sync_copy(part_sh.at[pl.ds(k * HIST + s * STRIPE, STRIPE)],
                        red_v.at[pl.ds(k * STRIPE, STRIPE)])
    for j in range(STRIPE // 16):
        tot = red_v[pl.ds(j * 16, 16)]
        for k in range(1, NSUB):
            tot = tot + red_v[pl.ds(k * STRIPE + j * 16, 16)]
        hist_v[pl.ds(j * 16, 16)] = tot

    @pl.when(((s < 8) & (c == 0)) | ((s >= 8) & (c == 1)))
    def _():
        pltpu.sync_copy(hist_v.at[pl.ds(0, STRIPE)],
                        deg_hbm.at[pl.ds(s * STRIPE, STRIPE)])


SUP = 1280
NSUP = E // SUP
CPS = SUP // CHUNK


@functools.partial(
    pl.kernel,
    mesh=_mesh,
    out_type=jax.ShapeDtypeStruct((NPAD, 16, 16), jnp.float32),
    scratch_types=[
        pltpu.VMEM((2, SUP), jnp.int32),
        pltpu.VMEM((2, CHUNK), jnp.int32),
        pltpu.VMEM((2, CHUNK), jnp.int32),
        pltpu.VMEM((2, CHUNK, 16), jnp.float32),
        pltpu.VMEM((ACC_R, 16), jnp.float32),
        pltpu.SemaphoreType.DMA,
        pltpu.SemaphoreType.DMA,
    ],
    compiler_params=_cp,
)
def _mp_sc(y16_hbm, eb_hbm, out_hbm,
           ebuf, gidx_v, ridx_v, msg_v, acc_v, sem0, sem1):
    c = lax.axis_index("c")
    s = lax.axis_index("s")
    lo = c * HALFN
    zero16 = jnp.zeros((16,), jnp.float32)
    iota16 = lax.iota(jnp.int32, 16)
    sems = (sem0, sem1)

    @pl.loop(0, ACC_R)
    def _(r):
        acc_v[r, :] = zero16

    def prep(t, b):
        for g in range(CHUNK // 16):
            s16 = ebuf[0, pl.ds(t * CHUNK + g * 16, 16)]
            d16 = ebuf[1, pl.ds(t * CHUNK + g * 16, 16)]
            local = d16 - lo
            ok = (local >= 0) & (local < HALFN)
            ridx_v[b, pl.ds(g * 16, 16)] = jnp.where(ok, local, HALFN)
            gidx_v[b, pl.ds(g * 16, 16)] = jnp.where(ok, s16 * 16 + s, -1)

    def fire(b):
        idx = plsc.Indices(gidx_v.at[b], ignored_value=-1)
        return pltpu.async_copy(y16_hbm.at[idx], msg_v.at[b], sems[b])

    def process(b):
        rows = [ridx_v[b, pl.ds(g * 16, 16)] for g in range(CHUNK // 16)]
        mrows = [iota16 + (g * 16) for g in range(CHUNK // 16)]

        @plsc.parallel_loop(0, 16, unroll=4)
        def _(j):
            rot = (iota16 + j) & 15
            for g in range(CHUNK // 16):
                vals = plsc.load_gather(msg_v.at[b], [mrows[g], rot])
                plsc.addupdate_scatter(acc_v, [rows[g], rot], vals)

    @pl.loop(0, NSUP)
    def _(u):
        pltpu.sync_copy(eb_hbm.at[u], ebuf)
        prep(0, 0)
        d = fire(0)
        for t in range(1, CPS):
            b = t % 2
            prep(t, b)
            d_next = fire(b)
            d.wait()
            process(1 - b)
            d = d_next
        d.wait()
        process((CPS - 1) % 2)

    pltpu.sync_copy(acc_v.at[pl.ds(0, HALFN)], out_hbm.at[pl.ds(lo, HALFN), s])


def _row_spec():
    return pl.BlockSpec((BR, D), lambda i: (i, 0))


def _col_spec():
    return pl.BlockSpec((BR, 1), lambda i: (i, 0))


def _w_spec():
    return pl.BlockSpec((D, D), lambda i: (0, 0))


def _vec_spec():
    return pl.BlockSpec((1, D), lambda i: (0, 0))


def _layer_norm(h, g, b):
    mu = jnp.mean(h, axis=-1, keepdims=True)
    var = jnp.mean((h - mu) ** 2, axis=-1, keepdims=True)
    return (h - mu) * lax.rsqrt(var + 1e-5) * g + b


def _tc1_body(x_ref, w_ref, deg_ref, y_ref, dinv_ref):
    deg = deg_ref[...] + 1.0
    dinv = jnp.where(deg > 0, lax.rsqrt(deg), 0.0)
    dinv_ref[...] = dinv
    y_ref[...] = dinv * jnp.dot(x_ref[...], w_ref[...],
                                preferred_element_type=jnp.float32)


_tc1 = pl.pallas_call(
    _tc1_body,
    grid=(GRID,),
    in_specs=[_row_spec(), _w_spec(), _col_spec()],
    out_specs=[_row_spec(), _col_spec()],
    out_shape=[jax.ShapeDtypeStruct((NPAD, D), jnp.float32),
               jax.ShapeDtypeStruct((NPAD, 1), jnp.float32)],
)


def _tc2_body(acc_ref, y_ref, dinv_ref, b_ref, g_ref, bln_ref, w_ref,
              h_ref, y1_ref):
    dinv = dinv_ref[...]
    h = jnp.maximum(dinv * (acc_ref[...] + y_ref[...]) + b_ref[...], 0.0)
    h_ref[...] = h
    t = _layer_norm(h, g_ref[...], bln_ref[...])
    y1_ref[...] = dinv * jnp.dot(t, w_ref[...],
                                 preferred_element_type=jnp.float32)


_tc2 = pl.pallas_call(
    _tc2_body,
    grid=(GRID,),
    in_specs=[_row_spec(), _row_spec(), _col_spec(), _vec_spec(), _vec_spec(),
              _vec_spec(), _w_spec()],
    out_specs=[_row_spec(), _row_spec()],
    out_shape=[jax.ShapeDtypeStruct((NPAD, D), jnp.float32),
               jax.ShapeDtypeStruct((NPAD, D), jnp.float32)],
)


def _tc3_body(acc_ref, y_ref, dinv_ref, b_ref, idn_ref, w_ref, y2_ref):
    dinv = dinv_ref[...]
    h = jnp.maximum(dinv * (acc_ref[...] + y_ref[...]) + b_ref[...], 0.0)
    h = h + idn_ref[...]
    y2_ref[...] = dinv * jnp.dot(h, w_ref[...],
                                 preferred_element_type=jnp.float32)


_tc3 = pl.pallas_call(
    _tc3_body,
    grid=(GRID,),
    in_specs=[_row_spec(), _row_spec(), _col_spec(), _vec_spec(), _row_spec(),
              _w_spec()],
    out_specs=_row_spec(),
    out_shape=jax.ShapeDtypeStruct((NPAD, D), jnp.float32),
)


def _tc4_body(acc_ref, y_ref, dinv_ref, b_ref, g_ref, bln_ref, o_ref):
    o = dinv_ref[...] * (acc_ref[...] + y_ref[...]) + b_ref[...]
    o_ref[...] = _layer_norm(o, g_ref[...], bln_ref[...])


_tc4 = pl.pallas_call(
    _tc4_body,
    grid=(GRID,),
    in_specs=[_row_spec(), _row_spec(), _col_spec(), _vec_spec(), _vec_spec(),
              _vec_spec()],
    out_specs=_row_spec(),
    out_shape=jax.ShapeDtypeStruct((NPAD, D), jnp.float32),
)


def _mp(y, eb):
    acc3 = _mp_sc(y.reshape(NPAD * 16, 16), eb)
    return acc3.reshape(NPAD, D)


def kernel(x, edge_index, W0, b0, W1, b1, W2, b2, ln0_g, ln0_b, fn_g, fn_b):
    src = edge_index[0]
    dst = edge_index[1]
    eb = jnp.stack([src.reshape(NSUP, SUP), dst.reshape(NSUP, SUP)], axis=1)
    xp = jnp.pad(x, ((0, NPAD - N), (0, 0)))

    deg = _deg_sc(dst)[:NPAD].reshape(NPAD, 1)
    y0, dinv = _tc1(xp, W0, deg)
    acc0 = _mp(y0, eb)
    h, y1 = _tc2(acc0, y0, dinv, b0.reshape(1, D), ln0_g.reshape(1, D),
                 ln0_b.reshape(1, D), W1)
    acc1 = _mp(y1, eb)
    y2 = _tc3(acc1, y1, dinv, b1.reshape(1, D), h, W2)
    acc2 = _mp(y2, eb)
    out = _tc4(acc2, y2, dinv, b2.reshape(1, D), fn_g.reshape(1, D),
               fn_b.reshape(1, D))
    return out[:N]


# speedup vs baseline: 2.2860x; 1.1486x over previous
import dataclasses
import functools

import jax
import jax.numpy as jnp
from jax import lax
from jax.experimental import pallas as pl
from jax.experimental.pallas import tpu as pltpu
from jax.experimental.pallas import tpu_sc as plsc

N = 10000
E = 160000
D = 256

NPAD = 10240
HALFN = 5120
ACC_R = 5136
CHUNK = 128
NCH = E // CHUNK
NSUB = 16
HIST = 10496
STRIPE = HIST // NSUB
BR = 1024
GRID = NPAD // BR

_mesh = plsc.VectorSubcoreMesh(core_axis_name="c", subcore_axis_name="s")

_cp = pltpu.CompilerParams()
if "needs_layout_passes" in pltpu.CompilerParams.__dataclass_fields__:
    _cp = dataclasses.replace(_cp, needs_layout_passes=False)
if "use_tc_tiling_on_sc" in pltpu.CompilerParams.__dataclass_fields__:
    _cp = dataclasses.replace(_cp, use_tc_tiling_on_sc=False)


@functools.partial(
    pl.kernel,
    mesh=_mesh,
    out_type=jax.ShapeDtypeStruct((HIST,), jnp.float32),
    scratch_types=[
        pltpu.VMEM((CHUNK,), jnp.int32),
        pltpu.VMEM((HIST,), jnp.float32),
        pltpu.VMEM((NSUB * STRIPE,), jnp.float32),
        pltpu.VMEM_SHARED((NSUB * HIST,), jnp.float32),
    ],
    compiler_params=_cp,
)
def _deg_sc(dst_hbm, deg_hbm, dst_v, hist_v, red_v, part_sh):
    c = lax.axis_index("c")
    s = lax.axis_index("s")
    one16 = jnp.full((16,), 1.0, jnp.float32)
    zero16 = jnp.zeros((16,), jnp.float32)

    @pl.loop(0, HIST // 16)
    def _(j):
        hist_v[pl.ds(j * 16, 16)] = zero16

    @pl.loop(0, (NCH + NSUB - 1) // NSUB)
    def _(i):
        k = i * NSUB + s

        @pl.when(k < NCH)
        def _():
            pltpu.sync_copy(dst_hbm.at[pl.ds(k * CHUNK, CHUNK)], dst_v)
            for j in range(CHUNK // 16):
                d16 = dst_v[pl.ds(j * 16, 16)]
                plsc.addupdate_scatter(hist_v, [d16], one16)

    pltpu.sync_copy(hist_v, part_sh.at[pl.ds(s * HIST, HIST)])
    plsc.subcore_barrier()
    for k in range(NSUB):
        pltpu.sync_copy(part_sh.at[pl.ds(k * HIST + s * STRIPE, STRIPE)],
                        red_v.at[pl.ds(k * STRIPE, STRIPE)])
    for j in range(STRIPE // 16):
        tot = red_v[pl.ds(j * 16, 16)]
        for k in range(1, NSUB):
            tot = tot + red_v[pl.ds(k * STRIPE + j * 16, 16)]
        hist_v[pl.ds(j * 16, 16)] = tot

    @pl.when(((s < 8) & (c == 0)) | ((s >= 8) & (c == 1)))
    def _():
        pltpu.sync_copy(hist_v.at[pl.ds(0, STRIPE)],
                        deg_hbm.at[pl.ds(s * STRIPE, STRIPE)])


SUP = 1280
NSUP = E // SUP
CPS = SUP // CHUNK


@functools.partial(
    pl.kernel,
    mesh=_mesh,
    out_type=jax.ShapeDtypeStruct((NPAD, 16, 16), jnp.float32),
    scratch_types=[
        pltpu.VMEM((2, SUP), jnp.int32),
        pltpu.VMEM((4, CHUNK), jnp.int32),
        pltpu.VMEM((4, CHUNK), jnp.int32),
        pltpu.VMEM((4, CHUNK, 16), jnp.float32),
        pltpu.VMEM((ACC_R, 16), jnp.float32),
        pltpu.SemaphoreType.DMA,
        pltpu.SemaphoreType.DMA,
        pltpu.SemaphoreType.DMA,
        pltpu.SemaphoreType.DMA,
    ],
    compiler_params=_cp,
)
def _mp_sc(y16_hbm, eb_hbm, out_hbm,
           ebuf, gidx_v, ridx_v, msg_v, acc_v, sem0, sem1, sem2, sem3):
    c = lax.axis_index("c")
    s = lax.axis_index("s")
    lo = c * HALFN
    zero16 = jnp.zeros((16,), jnp.float32)
    iota16 = lax.iota(jnp.int32, 16)
    sems = (sem0, sem1, sem2, sem3)

    @pl.loop(0, ACC_R)
    def _(r):
        acc_v[r, :] = zero16

    def prep(t, b):
        for g in range(CHUNK // 16):
            s16 = ebuf[0, pl.ds(t * CHUNK + g * 16, 16)]
            d16 = ebuf[1, pl.ds(t * CHUNK + g * 16, 16)]
            local = d16 - lo
            ok = (local >= 0) & (local < HALFN)
            ridx_v[b, pl.ds(g * 16, 16)] = jnp.where(ok, local, HALFN)
            gidx_v[b, pl.ds(g * 16, 16)] = jnp.where(ok, s16 * 16 + s, -1)

    def fire(b):
        idx = plsc.Indices(gidx_v.at[b], ignored_value=-1)
        return pltpu.async_copy(y16_hbm.at[idx], msg_v.at[b], sems[b])

    def process(b):
        rows = [ridx_v[b, pl.ds(g * 16, 16)] for g in range(CHUNK // 16)]
        mrows = [iota16 + (g * 16) for g in range(CHUNK // 16)]

        @plsc.parallel_loop(0, 16, unroll=4)
        def _(j):
            rot = (iota16 + j) & 15
            for g in range(CHUNK // 16):
                vals = plsc.load_gather(msg_v.at[b], [mrows[g], rot])
                plsc.addupdate_scatter(acc_v, [rows[g], rot], vals)

    DEPTH = 3

    @pl.loop(0, NSUP)
    def _(u):
        pltpu.sync_copy(eb_hbm.at[u], ebuf)
        ds = {}
        for t in range(DEPTH):
            prep(t, t % 4)
            ds[t] = fire(t % 4)
        for t in range(DEPTH, CPS):
            b = t % 4
            prep(t, b)
            ds[t] = fire(b)
            ds[t - DEPTH].wait()
            process((t - DEPTH) % 4)
        for t in range(CPS - DEPTH, CPS):
            ds[t].wait()
            process(t % 4)

    pltpu.sync_copy(acc_v.at[pl.ds(0, HALFN)], out_hbm.at[pl.ds(lo, HALFN), s])


def _row_spec():
    return pl.BlockSpec((BR, D), lambda i: (i, 0))


def _col_spec():
    return pl.BlockSpec((BR, 1), lambda i: (i, 0))


def _w_spec():
    return pl.BlockSpec((D, D), lambda i: (0, 0))


def _vec_spec():
    return pl.BlockSpec((1, D), lambda i: (0, 0))


def _layer_norm(h, g, b):
    mu = jnp.mean(h, axis=-1, keepdims=True)
    var = jnp.mean((h - mu) ** 2, axis=-1, keepdims=True)
    return (h - mu) * lax.rsqrt(var + 1e-5) * g + b


def _tc1_body(x_ref, w_ref, deg_ref, y_ref, dinv_ref):
    deg = deg_ref[...] + 1.0
    dinv = jnp.where(deg > 0, lax.rsqrt(deg), 0.0)
    dinv_ref[...] = dinv
    y_ref[...] = dinv * jnp.dot(x_ref[...], w_ref[...],
                                preferred_element_type=jnp.float32)


_tc1 = pl.pallas_call(
    _tc1_body,
    grid=(GRID,),
    in_specs=[_row_spec(), _w_spec(), _col_spec()],
    out_specs=[_row_spec(), _col_spec()],
    out_shape=[jax.ShapeDtypeStruct((NPAD, D), jnp.float32),
               jax.ShapeDtypeStruct((NPAD, 1), jnp.float32)],
)


def _tc2_body(acc_ref, y_ref, dinv_ref, b_ref, g_ref, bln_ref, w_ref,
              h_ref, y1_ref):
    dinv = dinv_ref[...]
    h = jnp.maximum(dinv * (acc_ref[...] + y_ref[...]) + b_ref[...], 0.0)
    h_ref[...] = h
    t = _layer_norm(h, g_ref[...], bln_ref[...])
    y1_ref[...] = dinv * jnp.dot(t, w_ref[...],
                                 preferred_element_type=jnp.float32)


_tc2 = pl.pallas_call(
    _tc2_body,
    grid=(GRID,),
    in_specs=[_row_spec(), _row_spec(), _col_spec(), _vec_spec(), _vec_spec(),
              _vec_spec(), _w_spec()],
    out_specs=[_row_spec(), _row_spec()],
    out_shape=[jax.ShapeDtypeStruct((NPAD, D), jnp.float32),
               jax.ShapeDtypeStruct((NPAD, D), jnp.float32)],
)


def _tc3_body(acc_ref, y_ref, dinv_ref, b_ref, idn_ref, w_ref, y2_ref):
    dinv = dinv_ref[...]
    h = jnp.maximum(dinv * (acc_ref[...] + y_ref[...]) + b_ref[...], 0.0)
    h = h + idn_ref[...]
    y2_ref[...] = dinv * jnp.dot(h, w_ref[...],
                                 preferred_element_type=jnp.float32)


_tc3 = pl.pallas_call(
    _tc3_body,
    grid=(GRID,),
    in_specs=[_row_spec(), _row_spec(), _col_spec(), _vec_spec(), _row_spec(),
              _w_spec()],
    out_specs=_row_spec(),
    out_shape=jax.ShapeDtypeStruct((NPAD, D), jnp.float32),
)


def _tc4_body(acc_ref, y_ref, dinv_ref, b_ref, g_ref, bln_ref, o_ref):
    o = dinv_ref[...] * (acc_ref[...] + y_ref[...]) + b_ref[...]
    o_ref[...] = _layer_norm(o, g_ref[...], bln_ref[...])


_tc4 = pl.pallas_call(
    _tc4_body,
    grid=(GRID,),
    in_specs=[_row_spec(), _row_spec(), _col_spec(), _vec_spec(), _vec_spec(),
              _vec_spec()],
    out_specs=_row_spec(),
    out_shape=jax.ShapeDtypeStruct((NPAD, D), jnp.float32),
)


def _mp(y, eb):
    acc3 = _mp_sc(y.reshape(NPAD * 16, 16), eb)
    return acc3.reshape(NPAD, D)


def kernel(x, edge_index, W0, b0, W1, b1, W2, b2, ln0_g, ln0_b, fn_g, fn_b):
    src = edge_index[0]
    dst = edge_index[1]
    eb = jnp.stack([src.reshape(NSUP, SUP), dst.reshape(NSUP, SUP)], axis=1)
    xp = jnp.pad(x, ((0, NPAD - N), (0, 0)))

    deg = _deg_sc(dst)[:NPAD].reshape(NPAD, 1)
    y0, dinv = _tc1(xp, W0, deg)
    acc0 = _mp(y0, eb)
    h, y1 = _tc2(acc0, y0, dinv, b0.reshape(1, D), ln0_g.reshape(1, D),
                 ln0_b.reshape(1, D), W1)
    acc1 = _mp(y1, eb)
    y2 = _tc3(acc1, y1, dinv, b1.reshape(1, D), h, W2)
    acc2 = _mp(y2, eb)
    out = _tc4(acc2, y2, dinv, b2.reshape(1, D), fn_g.reshape(1, D),
               fn_b.reshape(1, D))
    return out[:N]


# speedup vs baseline: 2.3502x; 1.0281x over previous
import dataclasses
import functools

import jax
import jax.numpy as jnp
from jax import lax
from jax.experimental import pallas as pl
from jax.experimental.pallas import tpu as pltpu
from jax.experimental.pallas import tpu_sc as plsc

N = 10000
E = 160000
D = 256

NPAD = 10240
HALFN = 5120
ACC_R = 5136
CHUNK = 128
NCH = E // CHUNK
NSUB = 16
HIST = 10496
STRIPE = HIST // NSUB
BR = 1024
GRID = NPAD // BR

_mesh = plsc.VectorSubcoreMesh(core_axis_name="c", subcore_axis_name="s")

_cp = pltpu.CompilerParams()
if "needs_layout_passes" in pltpu.CompilerParams.__dataclass_fields__:
    _cp = dataclasses.replace(_cp, needs_layout_passes=False)
if "use_tc_tiling_on_sc" in pltpu.CompilerParams.__dataclass_fields__:
    _cp = dataclasses.replace(_cp, use_tc_tiling_on_sc=False)


@functools.partial(
    pl.kernel,
    mesh=_mesh,
    out_type=jax.ShapeDtypeStruct((HIST,), jnp.float32),
    scratch_types=[
        pltpu.VMEM((CHUNK,), jnp.int32),
        pltpu.VMEM((HIST,), jnp.float32),
        pltpu.VMEM((NSUB * STRIPE,), jnp.float32),
        pltpu.VMEM_SHARED((NSUB * HIST,), jnp.float32),
    ],
    compiler_params=_cp,
)
def _deg_sc(dst_hbm, deg_hbm, dst_v, hist_v, red_v, part_sh):
    c = lax.axis_index("c")
    s = lax.axis_index("s")
    one16 = jnp.full((16,), 1.0, jnp.float32)
    zero16 = jnp.zeros((16,), jnp.float32)

    @pl.loop(0, HIST // 16)
    def _(j):
        hist_v[pl.ds(j * 16, 16)] = zero16

    @pl.loop(0, (NCH + NSUB - 1) // NSUB)
    def _(i):
        k = i * NSUB + s

        @pl.when(k < NCH)
        def _():
            pltpu.sync_copy(dst_hbm.at[pl.ds(k * CHUNK, CHUNK)], dst_v)
            for j in range(CHUNK // 16):
                d16 = dst_v[pl.ds(j * 16, 16)]
                plsc.addupdate_scatter(hist_v, [d16], one16)

    pltpu.sync_copy(hist_v, part_sh.at[pl.ds(s * HIST, HIST)])
    plsc.subcore_barrier()
    for k in range(NSUB):
        pltpu.sync_copy(part_sh.at[pl.ds(k * HIST + s * STRIPE, STRIPE)],
                        red_v.at[pl.ds(k * STRIPE, STRIPE)])
    for j in range(STRIPE // 16):
        tot = red_v[pl.ds(j * 16, 16)]
        for k in range(1, NSUB):
            tot = tot + red_v[pl.ds(k * STRIPE + j * 16, 16)]
        hist_v[pl.ds(j * 16, 16)] = tot

    @pl.when(((s < 8) & (c == 0)) | ((s >= 8) & (c == 1)))
    def _():
        pltpu.sync_copy(hist_v.at[pl.ds(0, STRIPE)],
                        deg_hbm.at[pl.ds(s * STRIPE, STRIPE)])


SUP = 1280
NSUP = E // SUP
CPS = SUP // CHUNK


@functools.partial(
    pl.kernel,
    mesh=_mesh,
    out_type=jax.ShapeDtypeStruct((NPAD, 16, 16), jnp.float32),
    scratch_types=[
        pltpu.VMEM((2, SUP), jnp.int32),
        pltpu.VMEM((6, CHUNK), jnp.int32),
        pltpu.VMEM((6, CHUNK), jnp.int32),
        pltpu.VMEM((6, CHUNK, 16), jnp.float32),
        pltpu.VMEM((ACC_R, 16), jnp.float32),
        pltpu.SemaphoreType.DMA,
        pltpu.SemaphoreType.DMA,
        pltpu.SemaphoreType.DMA,
        pltpu.SemaphoreType.DMA,
        pltpu.SemaphoreType.DMA,
        pltpu.SemaphoreType.DMA,
    ],
    compiler_params=_cp,
)
def _mp_sc(y16_hbm, eb_hbm, out_hbm,
           ebuf, gidx_v, ridx_v, msg_v, acc_v,
           sem0, sem1, sem2, sem3, sem4, sem5):
    c = lax.axis_index("c")
    s = lax.axis_index("s")
    lo = c * HALFN
    zero16 = jnp.zeros((16,), jnp.float32)
    iota16 = lax.iota(jnp.int32, 16)
    sems = (sem0, sem1, sem2, sem3, sem4, sem5)

    @pl.loop(0, ACC_R)
    def _(r):
        acc_v[r, :] = zero16

    def prep(t, b):
        for g in range(CHUNK // 16):
            s16 = ebuf[0, pl.ds(t * CHUNK + g * 16, 16)]
            d16 = ebuf[1, pl.ds(t * CHUNK + g * 16, 16)]
            local = d16 - lo
            ok = (local >= 0) & (local < HALFN)
            ridx_v[b, pl.ds(g * 16, 16)] = jnp.where(ok, local, HALFN)
            gidx_v[b, pl.ds(g * 16, 16)] = jnp.where(ok, s16 * 16 + s, -1)

    def fire(b):
        idx = plsc.Indices(gidx_v.at[b], ignored_value=-1)
        return pltpu.async_copy(y16_hbm.at[idx], msg_v.at[b], sems[b])

    def process(b):
        rows = [ridx_v[b, pl.ds(g * 16, 16)] for g in range(CHUNK // 16)]
        mrows = [iota16 + (g * 16) for g in range(CHUNK // 16)]

        @plsc.parallel_loop(0, 16, unroll=4)
        def _(j):
            rot = (iota16 + j) & 15
            for g in range(CHUNK // 16):
                vals = plsc.load_gather(msg_v.at[b], [mrows[g], rot])
                plsc.addupdate_scatter(acc_v, [rows[g], rot], vals)

    DEPTH = 5

    @pl.loop(0, NSUP)
    def _(u):
        pltpu.sync_copy(eb_hbm.at[u], ebuf)
        ds = {}
        for t in range(DEPTH):
            prep(t, t % 6)
            ds[t] = fire(t % 6)
        for t in range(DEPTH, CPS):
            b = t % 6
            prep(t, b)
            ds[t] = fire(b)
            ds[t - DEPTH].wait()
            process((t - DEPTH) % 6)
        for t in range(CPS - DEPTH, CPS):
            ds[t].wait()
            process(t % 6)

    pltpu.sync_copy(acc_v.at[pl.ds(0, HALFN)], out_hbm.at[pl.ds(lo, HALFN), s])


def _row_spec():
    return pl.BlockSpec((BR, D), lambda i: (i, 0))


def _col_spec():
    return pl.BlockSpec((BR, 1), lambda i: (i, 0))


def _w_spec():
    return pl.BlockSpec((D, D), lambda i: (0, 0))


def _vec_spec():
    return pl.BlockSpec((1, D), lambda i: (0, 0))


def _layer_norm(h, g, b):
    mu = jnp.mean(h, axis=-1, keepdims=True)
    var = jnp.mean((h - mu) ** 2, axis=-1, keepdims=True)
    return (h - mu) * lax.rsqrt(var + 1e-5) * g + b


def _tc1_body(x_ref, w_ref, deg_ref, y_ref, dinv_ref):
    deg = deg_ref[...] + 1.0
    dinv = jnp.where(deg > 0, lax.rsqrt(deg), 0.0)
    dinv_ref[...] = dinv
    y_ref[...] = dinv * jnp.dot(x_ref[...], w_ref[...],
                                preferred_element_type=jnp.float32)


_tc1 = pl.pallas_call(
    _tc1_body,
    grid=(GRID,),
    in_specs=[_row_spec(), _w_spec(), _col_spec()],
    out_specs=[_row_spec(), _col_spec()],
    out_shape=[jax.ShapeDtypeStruct((NPAD, D), jnp.float32),
               jax.ShapeDtypeStruct((NPAD, 1), jnp.float32)],
)


def _tc2_body(acc_ref, y_ref, dinv_ref, b_ref, g_ref, bln_ref, w_ref,
              h_ref, y1_ref):
    dinv = dinv_ref[...]
    h = jnp.maximum(dinv * (acc_ref[...] + y_ref[...]) + b_ref[...], 0.0)
    h_ref[...] = h
    t = _layer_norm(h, g_ref[...], bln_ref[...])
    y1_ref[...] = dinv * jnp.dot(t, w_ref[...],
                                 preferred_element_type=jnp.float32)


_tc2 = pl.pallas_call(
    _tc2_body,
    grid=(GRID,),
    in_specs=[_row_spec(), _row_spec(), _col_spec(), _vec_spec(), _vec_spec(),
              _vec_spec(), _w_spec()],
    out_specs=[_row_spec(), _row_spec()],
    out_shape=[jax.ShapeDtypeStruct((NPAD, D), jnp.float32),
               jax.ShapeDtypeStruct((NPAD, D), jnp.float32)],
)


def _tc3_body(acc_ref, y_ref, dinv_ref, b_ref, idn_ref, w_ref, y2_ref):
    dinv = dinv_ref[...]
    h = jnp.maximum(dinv * (acc_ref[...] + y_ref[...]) + b_ref[...], 0.0)
    h = h + idn_ref[...]
    y2_ref[...] = dinv * jnp.dot(h, w_ref[...],
                                 preferred_element_type=jnp.float32)


_tc3 = pl.pallas_call(
    _tc3_body,
    grid=(GRID,),
    in_specs=[_row_spec(), _row_spec(), _col_spec(), _vec_spec(), _row_spec(),
              _w_spec()],
    out_specs=_row_spec(),
    out_shape=jax.ShapeDtypeStruct((NPAD, D), jnp.float32),
)


def _tc4_body(acc_ref, y_ref, dinv_ref, b_ref, g_ref, bln_ref, o_ref):
    o = dinv_ref[...] * (acc_ref[...] + y_ref[...]) + b_ref[...]
    o_ref[...] = _layer_norm(o, g_ref[...], bln_ref[...])


_tc4 = pl.pallas_call(
    _tc4_body,
    grid=(GRID,),
    in_specs=[_row_spec(), _row_spec(), _col_spec(), _vec_spec(), _vec_spec(),
              _vec_spec()],
    out_specs=_row_spec(),
    out_shape=jax.ShapeDtypeStruct((NPAD, D), jnp.float32),
)


def _mp(y, eb):
    acc3 = _mp_sc(y.reshape(NPAD * 16, 16), eb)
    return acc3.reshape(NPAD, D)


def kernel(x, edge_index, W0, b0, W1, b1, W2, b2, ln0_g, ln0_b, fn_g, fn_b):
    src = edge_index[0]
    dst = edge_index[1]
    eb = jnp.stack([src.reshape(NSUP, SUP), dst.reshape(NSUP, SUP)], axis=1)
    xp = jnp.pad(x, ((0, NPAD - N), (0, 0)))

    deg = _deg_sc(dst)[:NPAD].reshape(NPAD, 1)
    y0, dinv = _tc1(xp, W0, deg)
    acc0 = _mp(y0, eb)
    h, y1 = _tc2(acc0, y0, dinv, b0.reshape(1, D), ln0_g.reshape(1, D),
                 ln0_b.reshape(1, D), W1)
    acc1 = _mp(y1, eb)
    y2 = _tc3(acc1, y1, dinv, b1.reshape(1, D), h, W2)
    acc2 = _mp(y2, eb)
    out = _tc4(acc2, y2, dinv, b2.reshape(1, D), fn_g.reshape(1, D),
               fn_b.reshape(1, D))
    return out[:N]


# speedup vs baseline: 2.3812x; 1.0132x over previous
import dataclasses
import functools

import jax
import jax.numpy as jnp
from jax import lax
from jax.experimental import pallas as pl
from jax.experimental.pallas import tpu as pltpu
from jax.experimental.pallas import tpu_sc as plsc

N = 10000
E = 160000
D = 256

NPAD = 10240
HALFN = 5120
ACC_R = 5136
CHUNK = 128
NCH = E // CHUNK
NSUB = 16
HIST = 10496
STRIPE = HIST // NSUB
BR = 1024
GRID = NPAD // BR

_mesh = plsc.VectorSubcoreMesh(core_axis_name="c", subcore_axis_name="s")

_cp = pltpu.CompilerParams()
if "needs_layout_passes" in pltpu.CompilerParams.__dataclass_fields__:
    _cp = dataclasses.replace(_cp, needs_layout_passes=False)
if "use_tc_tiling_on_sc" in pltpu.CompilerParams.__dataclass_fields__:
    _cp = dataclasses.replace(_cp, use_tc_tiling_on_sc=False)


@functools.partial(
    pl.kernel,
    mesh=_mesh,
    out_type=jax.ShapeDtypeStruct((HIST,), jnp.float32),
    scratch_types=[
        pltpu.VMEM((CHUNK,), jnp.int32),
        pltpu.VMEM((HIST,), jnp.float32),
        pltpu.VMEM((NSUB * STRIPE,), jnp.float32),
        pltpu.VMEM_SHARED((NSUB * HIST,), jnp.float32),
    ],
    compiler_params=_cp,
)
def _deg_sc(dst_hbm, deg_hbm, dst_v, hist_v, red_v, part_sh):
    c = lax.axis_index("c")
    s = lax.axis_index("s")
    one16 = jnp.full((16,), 1.0, jnp.float32)
    zero16 = jnp.zeros((16,), jnp.float32)

    @pl.loop(0, HIST // 16)
    def _(j):
        hist_v[pl.ds(j * 16, 16)] = zero16

    @pl.loop(0, (NCH + NSUB - 1) // NSUB)
    def _(i):
        k = i * NSUB + s

        @pl.when(k < NCH)
        def _():
            pltpu.sync_copy(dst_hbm.at[pl.ds(k * CHUNK, CHUNK)], dst_v)
            for j in range(CHUNK // 16):
                d16 = dst_v[pl.ds(j * 16, 16)]
                plsc.addupdate_scatter(hist_v, [d16], one16)

    pltpu.sync_copy(hist_v, part_sh.at[pl.ds(s * HIST, HIST)])
    plsc.subcore_barrier()
    for k in range(NSUB):
        pltpu.sync_copy(part_sh.at[pl.ds(k * HIST + s * STRIPE, STRIPE)],
                        red_v.at[pl.ds(k * STRIPE, STRIPE)])
    for j in range(STRIPE // 16):
        tot = red_v[pl.ds(j * 16, 16)]
        for k in range(1, NSUB):
            tot = tot + red_v[pl.ds(k * STRIPE + j * 16, 16)]
        hist_v[pl.ds(j * 16, 16)] = tot

    @pl.when(((s < 8) & (c == 0)) | ((s >= 8) & (c == 1)))
    def _():
        pltpu.sync_copy(hist_v.at[pl.ds(0, STRIPE)],
                        deg_hbm.at[pl.ds(s * STRIPE, STRIPE)])


SUP = 3200
NSUP = E // SUP
CPS = SUP // CHUNK


@functools.partial(
    pl.kernel,
    mesh=_mesh,
    out_type=jax.ShapeDtypeStruct((NPAD, 16, 16), jnp.float32),
    scratch_types=[
        pltpu.VMEM((2, SUP), jnp.int32),
        pltpu.VMEM((6, CHUNK), jnp.int32),
        pltpu.VMEM((6, CHUNK), jnp.int32),
        pltpu.VMEM((6, CHUNK, 16), jnp.float32),
        pltpu.VMEM((ACC_R, 16), jnp.float32),
        pltpu.SemaphoreType.DMA,
        pltpu.SemaphoreType.DMA,
        pltpu.SemaphoreType.DMA,
        pltpu.SemaphoreType.DMA,
        pltpu.SemaphoreType.DMA,
        pltpu.SemaphoreType.DMA,
    ],
    compiler_params=_cp,
)
def _mp_sc(y16_hbm, eb_hbm, out_hbm,
           ebuf, gidx_v, ridx_v, msg_v, acc_v,
           sem0, sem1, sem2, sem3, sem4, sem5):
    c = lax.axis_index("c")
    s = lax.axis_index("s")
    lo = c * HALFN
    zero16 = jnp.zeros((16,), jnp.float32)
    iota16 = lax.iota(jnp.int32, 16)
    sems = (sem0, sem1, sem2, sem3, sem4, sem5)

    @pl.loop(0, ACC_R)
    def _(r):
        acc_v[r, :] = zero16

    def prep(t, b):
        for g in range(CHUNK // 16):
            s16 = ebuf[0, pl.ds(t * CHUNK + g * 16, 16)]
            d16 = ebuf[1, pl.ds(t * CHUNK + g * 16, 16)]
            local = d16 - lo
            ok = (local >= 0) & (local < HALFN)
            ridx_v[b, pl.ds(g * 16, 16)] = jnp.where(ok, local, HALFN)
            gidx_v[b, pl.ds(g * 16, 16)] = jnp.where(ok, s16 * 16 + s, -1)

    def fire(b):
        idx = plsc.Indices(gidx_v.at[b], ignored_value=-1)
        return pltpu.async_copy(y16_hbm.at[idx], msg_v.at[b], sems[b])

    def process(b):
        rows = [ridx_v[b, pl.ds(g * 16, 16)] for g in range(CHUNK // 16)]
        mrows = [iota16 + (g * 16) for g in range(CHUNK // 16)]

        @plsc.parallel_loop(0, 16, unroll=4)
        def _(j):
            rot = (iota16 + j) & 15
            for g in range(CHUNK // 16):
                vals = plsc.load_gather(msg_v.at[b], [mrows[g], rot])
                plsc.addupdate_scatter(acc_v, [rows[g], rot], vals)

    DEPTH = 5

    @pl.loop(0, NSUP)
    def _(u):
        pltpu.sync_copy(eb_hbm.at[u], ebuf)
        ds = {}
        for t in range(DEPTH):
            prep(t, t % 6)
            ds[t] = fire(t % 6)
        for t in range(DEPTH, CPS):
            b = t % 6
            prep(t, b)
            ds[t] = fire(b)
            ds[t - DEPTH].wait()
            process((t - DEPTH) % 6)
        for t in range(CPS - DEPTH, CPS):
            ds[t].wait()
            process(t % 6)

    pltpu.sync_copy(acc_v.at[pl.ds(0, HALFN)], out_hbm.at[pl.ds(lo, HALFN), s])


def _row_spec():
    return pl.BlockSpec((BR, D), lambda i: (i, 0))


def _col_spec():
    return pl.BlockSpec((BR, 1), lambda i: (i, 0))


def _w_spec():
    return pl.BlockSpec((D, D), lambda i: (0, 0))


def _vec_spec():
    return pl.BlockSpec((1, D), lambda i: (0, 0))


def _layer_norm(h, g, b):
    mu = jnp.mean(h, axis=-1, keepdims=True)
    var = jnp.mean((h - mu) ** 2, axis=-1, keepdims=True)
    return (h - mu) * lax.rsqrt(var + 1e-5) * g + b


def _tc1_body(x_ref, w_ref, deg_ref, y_ref, dinv_ref):
    deg = deg_ref[...] + 1.0
    dinv = jnp.where(deg > 0, lax.rsqrt(deg), 0.0)
    dinv_ref[...] = dinv
    y_ref[...] = dinv * jnp.dot(x_ref[...], w_ref[...],
                                preferred_element_type=jnp.float32)


_tc1 = pl.pallas_call(
    _tc1_body,
    grid=(GRID,),
    in_specs=[_row_spec(), _w_spec(), _col_spec()],
    out_specs=[_row_spec(), _col_spec()],
    out_shape=[jax.ShapeDtypeStruct((NPAD, D), jnp.float32),
               jax.ShapeDtypeStruct((NPAD, 1), jnp.float32)],
)


def _tc2_body(acc_ref, y_ref, dinv_ref, b_ref, g_ref, bln_ref, w_ref,
              h_ref, y1_ref):
    dinv = dinv_ref[...]
    h = jnp.maximum(dinv * (acc_ref[...] + y_ref[...]) + b_ref[...], 0.0)
    h_ref[...] = h
    t = _layer_norm(h, g_ref[...], bln_ref[...])
    y1_ref[...] = dinv * jnp.dot(t, w_ref[...],
                                 preferred_element_type=jnp.float32)


_tc2 = pl.pallas_call(
    _tc2_body,
    grid=(GRID,),
    in_specs=[_row_spec(), _row_spec(), _col_spec(), _vec_spec(), _vec_spec(),
              _vec_spec(), _w_spec()],
    out_specs=[_row_spec(), _row_spec()],
    out_shape=[jax.ShapeDtypeStruct((NPAD, D), jnp.float32),
               jax.ShapeDtypeStruct((NPAD, D), jnp.float32)],
)


def _tc3_body(acc_ref, y_ref, dinv_ref, b_ref, idn_ref, w_ref, y2_ref):
    dinv = dinv_ref[...]
    h = jnp.maximum(dinv * (acc_ref[...] + y_ref[...]) + b_ref[...], 0.0)
    h = h + idn_ref[...]
    y2_ref[...] = dinv * jnp.dot(h, w_ref[...],
                                 preferred_element_type=jnp.float32)


_tc3 = pl.pallas_call(
    _tc3_body,
    grid=(GRID,),
    in_specs=[_row_spec(), _row_spec(), _col_spec(), _vec_spec(), _row_spec(),
              _w_spec()],
    out_specs=_row_spec(),
    out_shape=jax.ShapeDtypeStruct((NPAD, D), jnp.float32),
)


def _tc4_body(acc_ref, y_ref, dinv_ref, b_ref, g_ref, bln_ref, o_ref):
    o = dinv_ref[...] * (acc_ref[...] + y_ref[...]) + b_ref[...]
    o_ref[...] = _layer_norm(o, g_ref[...], bln_ref[...])


_tc4 = pl.pallas_call(
    _tc4_body,
    grid=(GRID,),
    in_specs=[_row_spec(), _row_spec(), _col_spec(), _vec_spec(), _vec_spec(),
              _vec_spec()],
    out_specs=_row_spec(),
    out_shape=jax.ShapeDtypeStruct((NPAD, D), jnp.float32),
)


def _mp(y, eb):
    acc3 = _mp_sc(y.reshape(NPAD * 16, 16), eb)
    return acc3.reshape(NPAD, D)


def kernel(x, edge_index, W0, b0, W1, b1, W2, b2, ln0_g, ln0_b, fn_g, fn_b):
    src = edge_index[0]
    dst = edge_index[1]
    eb = jnp.stack([src.reshape(NSUP, SUP), dst.reshape(NSUP, SUP)], axis=1)
    xp = jnp.pad(x, ((0, NPAD - N), (0, 0)))

    deg = _deg_sc(dst)[:NPAD].reshape(NPAD, 1)
    y0, dinv = _tc1(xp, W0, deg)
    acc0 = _mp(y0, eb)
    h, y1 = _tc2(acc0, y0, dinv, b0.reshape(1, D), ln0_g.reshape(1, D),
                 ln0_b.reshape(1, D), W1)
    acc1 = _mp(y1, eb)
    y2 = _tc3(acc1, y1, dinv, b1.reshape(1, D), h, W2)
    acc2 = _mp(y2, eb)
    out = _tc4(acc2, y2, dinv, b2.reshape(1, D), fn_g.reshape(1, D),
               fn_b.reshape(1, D))
    return out[:N]
